# Initial kernel scaffold; baseline (speedup 1.0000x reference)
import jax
import jax.numpy as jnp
from jax.experimental import pallas as pl


def kernel(x, edge_index, edge_attr, W_in, b_in, g_in, be_in, m_in, v_in, W_e, b_e, g_e, be_e, m_e, v_e, Wf1, bf1, Ws1, bs1, Wf2, bf2, Ws2, bs2, W_out, b_out):
    raise NotImplementedError("write your pallas kernel here")


# speedup vs baseline: 2.1134x; 2.1134x over previous
import functools

import jax
import jax.numpy as jnp
import numpy as np
from jax import lax
from jax.experimental import pallas as pl
from jax.experimental.pallas import tpu as pltpu
from jax.experimental.pallas import tpu_sc as plsc

N, E, D, DE, H, O = 10000, 320000, 128, 16, 32, 128

NC, NS = 2, 16
NW = NC * NS
CHUNK = 128
CPW = -(-E // (NW * CHUNK))
NCHUNKS = NW * CPW
E_PAD = NCHUNKS * CHUNK
N_TAB = N + 16
N_AGG = N + 16
CR = 624
CR_TAIL = N_AGG - NS * CR
OUT_TAIL = N - NS * CR

_LOG2_C = [
    -3.2407021416934514, 7.110035208934455, -7.44387313733697,
    5.723401325205929, -2.945206207987371, 0.961866323099821,
    -0.1802997712862884, 0.014778720761710662,
]
_LOG2_C[0] -= sum(_LOG2_C)
LN2 = 0.6931471805599453


def _log2_poly(u):
    r = jnp.full_like(u, _LOG2_C[7])
    for c in (_LOG2_C[6], _LOG2_C[5], _LOG2_C[4], _LOG2_C[3],
              _LOG2_C[2], _LOG2_C[1], _LOG2_C[0]):
        r = r * u + c
    return r


def _softplus_sc(s):
    t = jnp.exp(-jnp.abs(s))
    return jnp.maximum(s, 0.0) + LN2 * _log2_poly(1.0 + t)


def _sigmoid_sc(f):
    t = jnp.exp(-jnp.abs(f))
    r = 1.0 / (1.0 + t)
    return jnp.where(f >= 0, r, t * r)


def _leaky(h):
    return jnp.where(h >= 0, h, 0.1 * h)


def _edge_pre_body(ea_ref, we_ref, be_ref, wcat_ref, ep1_ref, ep2_ref):
    e = _leaky(jnp.dot(ea_ref[...], we_ref[...],
                       preferred_element_type=jnp.float32) + be_ref[...])
    r = jnp.dot(e, wcat_ref[...], preferred_element_type=jnp.float32)
    ep1_ref[...] = r[:, :64]
    ep2_ref[...] = r[:, 64:]


def _edge_pre(ea_pad, we, be, wcat):
    BE = 2048
    grid = E_PAD // BE
    return pl.pallas_call(
        _edge_pre_body,
        grid=(grid,),
        in_specs=[
            pl.BlockSpec((BE, DE), lambda i: (i, 0)),
            pl.BlockSpec((DE, H), lambda i: (0, 0)),
            pl.BlockSpec((1, H), lambda i: (0, 0)),
            pl.BlockSpec((H, 128), lambda i: (0, 0)),
        ],
        out_specs=[
            pl.BlockSpec((BE, 2 * H), lambda i: (i, 0)),
            pl.BlockSpec((BE, 2 * H), lambda i: (i, 0)),
        ],
        out_shape=[
            jax.ShapeDtypeStruct((E_PAD, 2 * H), jnp.float32),
            jax.ShapeDtypeStruct((E_PAD, 2 * H), jnp.float32),
        ],
    )(ea_pad, we, be.reshape(1, H), wcat)


def _node_pre_body(x_ref, win_ref, bin_ref, wt_ref, bt_ref, h_ref, t_ref):
    h = _leaky(jnp.dot(x_ref[...], win_ref[...],
                       preferred_element_type=jnp.float32) + bin_ref[...])
    h_ref[...] = h
    t_ref[...] = jnp.dot(h, wt_ref[...],
                         preferred_element_type=jnp.float32) + bt_ref[...]


def _node_pre(x, win, bin_, wt, bt):
    BN = 2000
    grid = N // BN
    return pl.pallas_call(
        _node_pre_body,
        grid=(grid,),
        in_specs=[
            pl.BlockSpec((BN, D), lambda i: (i, 0)),
            pl.BlockSpec((D, H), lambda i: (0, 0)),
            pl.BlockSpec((1, H), lambda i: (0, 0)),
            pl.BlockSpec((H, 4 * H), lambda i: (0, 0)),
            pl.BlockSpec((1, 4 * H), lambda i: (0, 0)),
        ],
        out_specs=[
            pl.BlockSpec((BN, H), lambda i: (i, 0)),
            pl.BlockSpec((BN, 4 * H), lambda i: (i, 0)),
        ],
        out_shape=[
            jax.ShapeDtypeStruct((N, H), jnp.float32),
            jax.ShapeDtypeStruct((N, 4 * H), jnp.float32),
        ],
    )(x, win, bin_.reshape(1, H), wt, bt.reshape(1, 4 * H))


def _node_mid_body(h_ref, agg_ref, wt_ref, bt_ref, h1_ref, t_ref):
    h1 = h_ref[...] + agg_ref[0] + agg_ref[1]
    h1_ref[...] = h1
    t_ref[...] = jnp.dot(h1, wt_ref[...],
                         preferred_element_type=jnp.float32) + bt_ref[...]


def _node_mid(h, agg, wt, bt):
    BN = 2000
    grid = N // BN
    return pl.pallas_call(
        _node_mid_body,
        grid=(grid,),
        in_specs=[
            pl.BlockSpec((BN, H), lambda i: (i, 0)),
            pl.BlockSpec((NC, BN, H), lambda i: (0, i, 0)),
            pl.BlockSpec((H, 4 * H), lambda i: (0, 0)),
            pl.BlockSpec((1, 4 * H), lambda i: (0, 0)),
        ],
        out_specs=[
            pl.BlockSpec((BN, H), lambda i: (i, 0)),
            pl.BlockSpec((BN, 4 * H), lambda i: (i, 0)),
        ],
        out_shape=[
            jax.ShapeDtypeStruct((N, H), jnp.float32),
            jax.ShapeDtypeStruct((N, 4 * H), jnp.float32),
        ],
    )(h, agg, wt, bt.reshape(1, 4 * H))


def _node_out_body(h_ref, agg_ref, wout_ref, bout_ref, out_ref):
    h2 = h_ref[...] + agg_ref[0] + agg_ref[1]
    out_ref[...] = jnp.dot(h2, wout_ref[...],
                           preferred_element_type=jnp.float32) + bout_ref[...]


def _node_out(h, agg, wout, bout):
    BN = 2000
    grid = N // BN
    return pl.pallas_call(
        _node_out_body,
        grid=(grid,),
        in_specs=[
            pl.BlockSpec((BN, H), lambda i: (i, 0)),
            pl.BlockSpec((NC, BN, H), lambda i: (0, i, 0)),
            pl.BlockSpec((H, O), lambda i: (0, 0)),
            pl.BlockSpec((1, O), lambda i: (0, 0)),
        ],
        out_specs=pl.BlockSpec((BN, O), lambda i: (i, 0)),
        out_shape=jax.ShapeDtypeStruct((N, O), jnp.float32),
    )(h, agg, wout, bout.reshape(1, O))


def _edge_pass_body(t_hbm, ep_hbm, idx_hbm, out_hbm,
                    idx_v, td_v, ts_v, e_v, m_v, z_v, agg_sh, sem_d, sem_s):
    cid = lax.axis_index("c")
    sid = lax.axis_index("s")
    wid = sid * NC + cid

    zero16 = jnp.zeros((16,), jnp.float32)

    def zero_body(i, _):
        z_v[i, pl.ds(0, 16)] = zero16
        z_v[i, pl.ds(16, 16)] = zero16
        return 0

    lax.fori_loop(0, CR, zero_body, 0, unroll=4)
    pltpu.sync_copy(z_v, agg_sh.at[pl.ds(sid * CR, CR)])

    @pl.when(sid == 0)
    def _():
        pltpu.sync_copy(z_v.at[pl.ds(0, CR_TAIL)],
                        agg_sh.at[pl.ds(NS * CR, CR_TAIL)])

    plsc.subcore_barrier()

    def chunk_body(j, _):
        c = wid * CPW + j
        pltpu.sync_copy(idx_hbm.at[c], idx_v)
        g_d = pltpu.async_copy(t_hbm.at[idx_v.at[1]], td_v, sem_d)
        g_s = pltpu.async_copy(t_hbm.at[idx_v.at[0]], ts_v, sem_s)
        pltpu.sync_copy(ep_hbm.at[pl.ds(c * CHUNK, CHUNK)], e_v)
        g_d.wait()
        g_s.wait()

        def edge_body(i, _):
            f0 = td_v[i, pl.ds(0, 16)] + ts_v[i, pl.ds(64, 16)] + e_v[i, pl.ds(0, 16)]
            f1 = td_v[i, pl.ds(16, 16)] + ts_v[i, pl.ds(80, 16)] + e_v[i, pl.ds(16, 16)]
            s0 = td_v[i, pl.ds(32, 16)] + ts_v[i, pl.ds(96, 16)] + e_v[i, pl.ds(32, 16)]
            s1 = td_v[i, pl.ds(48, 16)] + ts_v[i, pl.ds(112, 16)] + e_v[i, pl.ds(48, 16)]
            m_v[i, pl.ds(0, 16)] = _sigmoid_sc(f0) * _softplus_sc(s0)
            m_v[i, pl.ds(16, 16)] = _sigmoid_sc(f1) * _softplus_sc(s1)
            return 0

        lax.fori_loop(0, CHUNK, edge_body, 0)
        pltpu.sync_copy(m_v, agg_sh.at[idx_v.at[1]], add=True)
        return 0

    lax.fori_loop(0, CPW, chunk_body, 0)
    plsc.subcore_barrier()

    pltpu.sync_copy(agg_sh.at[pl.ds(sid * CR, CR)], z_v)
    pltpu.sync_copy(z_v, out_hbm.at[cid, pl.ds(sid * CR, CR)])

    @pl.when(sid == 0)
    def _():
        pltpu.sync_copy(agg_sh.at[pl.ds(NS * CR, OUT_TAIL)],
                        z_v.at[pl.ds(0, OUT_TAIL)])
        pltpu.sync_copy(z_v.at[pl.ds(0, OUT_TAIL)],
                        out_hbm.at[cid, pl.ds(NS * CR, OUT_TAIL)])


def _edge_pass(t_pad, ep, idx):
    mesh = plsc.VectorSubcoreMesh(core_axis_name="c", subcore_axis_name="s",
                                  num_cores=NC, num_subcores=NS)
    f = functools.partial(
        pl.kernel,
        out_type=jax.ShapeDtypeStruct((NC, N, H), jnp.float32),
        mesh=mesh,
        scratch_types=[
            pltpu.VMEM((2, CHUNK), jnp.int32),
            pltpu.VMEM((CHUNK, 4 * H), jnp.float32),
            pltpu.VMEM((CHUNK, 4 * H), jnp.float32),
            pltpu.VMEM((CHUNK, 2 * H), jnp.float32),
            pltpu.VMEM((CHUNK, H), jnp.float32),
            pltpu.VMEM((CR, H), jnp.float32),
            pltpu.VMEM_SHARED((N_AGG, H), jnp.float32),
            pltpu.SemaphoreType.DMA,
            pltpu.SemaphoreType.DMA,
        ],
        compiler_params=pltpu.CompilerParams(use_tc_tiling_on_sc=False),
    )(_edge_pass_body)
    return f(t_pad, ep, idx)


def _fold_bn(w, b, g, be, m, v, eps=1e-5):
    scale = g / jnp.sqrt(v + eps)
    return w * scale[None, :], (b - m) * scale + be


def kernel(x, edge_index, edge_attr, W_in, b_in, g_in, be_in, m_in, v_in,
           W_e, b_e, g_e, be_e, m_e, v_e, Wf1, bf1, Ws1, bs1, Wf2, bf2,
           Ws2, bs2, W_out, b_out):
    win, bin_ = _fold_bn(W_in, b_in, g_in, be_in, m_in, v_in)
    we, be = _fold_bn(W_e, b_e, g_e, be_e, m_e, v_e)

    zeros2h = jnp.zeros((2 * H,), jnp.float32)
    wt1 = jnp.concatenate([Wf1[0:H], Ws1[0:H], Wf1[H:2 * H], Ws1[H:2 * H]],
                          axis=1)
    bt1 = jnp.concatenate([bf1, bs1, zeros2h])
    wt2 = jnp.concatenate([Wf2[0:H], Ws2[0:H], Wf2[H:2 * H], Ws2[H:2 * H]],
                          axis=1)
    bt2 = jnp.concatenate([bf2, bs2, zeros2h])
    wcat = jnp.concatenate([Wf1[2 * H:], Ws1[2 * H:], Wf2[2 * H:], Ws2[2 * H:]],
                           axis=1)

    src = edge_index[0]
    dst = edge_index[1]
    pad = E_PAD - E
    src_p = jnp.concatenate([src, jnp.zeros((pad,), jnp.int32)])
    dst_p = jnp.concatenate([dst, jnp.full((pad,), N, jnp.int32)])
    idx = jnp.stack([src_p, dst_p], 0).reshape(2, NCHUNKS, CHUNK).transpose(1, 0, 2)
    ea_pad = jnp.pad(edge_attr, ((0, pad), (0, 0)))

    ep1, ep2 = _edge_pre(ea_pad, we, be, wcat)

    h0, t1 = _node_pre(x, win, bin_, wt1, bt1)

    t1p = jnp.pad(t1, ((0, N_TAB - N), (0, 0)))
    agg1 = _edge_pass(t1p, ep1, idx)

    h1, t2 = _node_mid(h0, agg1, wt2, bt2)

    t2p = jnp.pad(t2, ((0, N_TAB - N), (0, 0)))
    agg2 = _edge_pass(t2p, ep2, idx)

    return _node_out(h1, agg2, W_out, b_out)


# speedup vs baseline: 3.5060x; 1.6589x over previous
import functools

import jax
import jax.numpy as jnp
from jax import lax
from jax.experimental import pallas as pl
from jax.experimental.pallas import tpu as pltpu
from jax.experimental.pallas import tpu_sc as plsc

N, E, D, DE, H, O = 10000, 320000, 128, 16, 32, 128

NC, NS = 2, 16
NW = NC * NS
CHUNK = 128
CPW = 80
NCHUNKS = NW * CPW
E_PAD = NCHUNKS * CHUNK
N_TAB = N + 16
N_AGG = N + 16
CR = 624
CR_TAIL = N_AGG - NS * CR
OUT_TAIL = N - NS * CR

_LN_C = [
    -1.9367697179748704, 3.5140872970008568, -2.440029762615309,
    1.1160900268329503, -0.28382684778232653, 0.030449004538698962,
]
_LN_C[0] -= sum(_LN_C)


def _ln_poly(u):
    r = jnp.full_like(u, _LN_C[5])
    for c in (_LN_C[4], _LN_C[3], _LN_C[2], _LN_C[1], _LN_C[0]):
        r = r * u + c
    return r


def _gate(f, s):
    sig = 1.0 / (1.0 + jnp.exp(-f))
    sp = jnp.maximum(s, 0.0) + _ln_poly(1.0 + jnp.exp(-jnp.abs(s)))
    return sig * sp


def _leaky(h):
    return jnp.where(h >= 0, h, 0.1 * h)


def _edge_pre_body(ea_ref, we_ref, be_ref, wcat_ref, ep1_ref, ep2_ref):
    e = _leaky(jnp.dot(ea_ref[...], we_ref[...],
                       preferred_element_type=jnp.float32) + be_ref[...])
    r = jnp.dot(e, wcat_ref[...], preferred_element_type=jnp.float32)
    ep1_ref[...] = r[:, :64]
    ep2_ref[...] = r[:, 64:]


def _edge_pre(ea_pad, we, be, wcat):
    BE = 2048
    grid = E_PAD // BE
    return pl.pallas_call(
        _edge_pre_body,
        grid=(grid,),
        in_specs=[
            pl.BlockSpec((BE, DE), lambda i: (i, 0)),
            pl.BlockSpec((DE, H), lambda i: (0, 0)),
            pl.BlockSpec((1, H), lambda i: (0, 0)),
            pl.BlockSpec((H, 128), lambda i: (0, 0)),
        ],
        out_specs=[
            pl.BlockSpec((BE, 2 * H), lambda i: (i, 0)),
            pl.BlockSpec((BE, 2 * H), lambda i: (i, 0)),
        ],
        out_shape=[
            jax.ShapeDtypeStruct((E_PAD, 2 * H), jnp.float32),
            jax.ShapeDtypeStruct((E_PAD, 2 * H), jnp.float32),
        ],
    )(ea_pad, we, be.reshape(1, H), wcat)


def _node_pre_body(x_ref, win_ref, bin_ref, wtd_ref, btd_ref, wts_ref,
                   h_ref, td_ref, ts_ref):
    h = _leaky(jnp.dot(x_ref[...], win_ref[...],
                       preferred_element_type=jnp.float32) + bin_ref[...])
    h_ref[...] = h
    td_ref[...] = jnp.dot(h, wtd_ref[...],
                          preferred_element_type=jnp.float32) + btd_ref[...]
    ts_ref[...] = jnp.dot(h, wts_ref[...], preferred_element_type=jnp.float32)


def _node_pre(x, win, bin_, wtd, btd, wts):
    BN = 2000
    grid = N // BN
    return pl.pallas_call(
        _node_pre_body,
        grid=(grid,),
        in_specs=[
            pl.BlockSpec((BN, D), lambda i: (i, 0)),
            pl.BlockSpec((D, H), lambda i: (0, 0)),
            pl.BlockSpec((1, H), lambda i: (0, 0)),
            pl.BlockSpec((H, 2 * H), lambda i: (0, 0)),
            pl.BlockSpec((1, 2 * H), lambda i: (0, 0)),
            pl.BlockSpec((H, 2 * H), lambda i: (0, 0)),
        ],
        out_specs=[
            pl.BlockSpec((BN, H), lambda i: (i, 0)),
            pl.BlockSpec((BN, 2 * H), lambda i: (i, 0)),
            pl.BlockSpec((BN, 2 * H), lambda i: (i, 0)),
        ],
        out_shape=[
            jax.ShapeDtypeStruct((N, H), jnp.float32),
            jax.ShapeDtypeStruct((N, 2 * H), jnp.float32),
            jax.ShapeDtypeStruct((N, 2 * H), jnp.float32),
        ],
    )(x, win, bin_.reshape(1, H), wtd, btd.reshape(1, 2 * H), wts)


def _node_mid_body(h_ref, agg_ref, wtd_ref, btd_ref, wts_ref,
                   h1_ref, td_ref, ts_ref):
    h1 = h_ref[...] + agg_ref[0] + agg_ref[1]
    h1_ref[...] = h1
    td_ref[...] = jnp.dot(h1, wtd_ref[...],
                          preferred_element_type=jnp.float32) + btd_ref[...]
    ts_ref[...] = jnp.dot(h1, wts_ref[...], preferred_element_type=jnp.float32)


def _node_mid(h, agg, wtd, btd, wts):
    BN = 2000
    grid = N // BN
    return pl.pallas_call(
        _node_mid_body,
        grid=(grid,),
        in_specs=[
            pl.BlockSpec((BN, H), lambda i: (i, 0)),
            pl.BlockSpec((NC, BN, H), lambda i: (0, i, 0)),
            pl.BlockSpec((H, 2 * H), lambda i: (0, 0)),
            pl.BlockSpec((1, 2 * H), lambda i: (0, 0)),
            pl.BlockSpec((H, 2 * H), lambda i: (0, 0)),
        ],
        out_specs=[
            pl.BlockSpec((BN, H), lambda i: (i, 0)),
            pl.BlockSpec((BN, 2 * H), lambda i: (i, 0)),
            pl.BlockSpec((BN, 2 * H), lambda i: (i, 0)),
        ],
        out_shape=[
            jax.ShapeDtypeStruct((N, H), jnp.float32),
            jax.ShapeDtypeStruct((N, 2 * H), jnp.float32),
            jax.ShapeDtypeStruct((N, 2 * H), jnp.float32),
        ],
    )(h, agg, wtd, btd.reshape(1, 2 * H), wts)


def _node_out_body(h_ref, agg_ref, wout_ref, bout_ref, out_ref):
    h2 = h_ref[...] + agg_ref[0] + agg_ref[1]
    out_ref[...] = jnp.dot(h2, wout_ref[...],
                           preferred_element_type=jnp.float32) + bout_ref[...]


def _node_out(h, agg, wout, bout):
    BN = 2000
    grid = N // BN
    return pl.pallas_call(
        _node_out_body,
        grid=(grid,),
        in_specs=[
            pl.BlockSpec((BN, H), lambda i: (i, 0)),
            pl.BlockSpec((NC, BN, H), lambda i: (0, i, 0)),
            pl.BlockSpec((H, O), lambda i: (0, 0)),
            pl.BlockSpec((1, O), lambda i: (0, 0)),
        ],
        out_specs=pl.BlockSpec((BN, O), lambda i: (i, 0)),
        out_shape=jax.ShapeDtypeStruct((N, O), jnp.float32),
    )(h, agg, wout, bout.reshape(1, O))


def _edge_pass_body(td_hbm, ts_hbm, ep_hbm, idx_hbm, out_hbm,
                    idx_v, gtd_v, gts_v, ge_v, m_v, z_v, agg_sh,
                    sem_td0, sem_td1, sem_ts0, sem_ts1, sem_e0, sem_e1):
    cid = lax.axis_index("c")
    sid = lax.axis_index("s")
    wid = sid * NC + cid
    sems = ((sem_td0, sem_ts0, sem_e0), (sem_td1, sem_ts1, sem_e1))

    pltpu.sync_copy(idx_hbm.at[pl.ds(wid * CPW, CPW)], idx_v)

    zero16 = jnp.zeros((16,), jnp.float32)

    def zero_body(i, _):
        z_v[i, pl.ds(0, 16)] = zero16
        z_v[i, pl.ds(16, 16)] = zero16
        return 0

    lax.fori_loop(0, CR, zero_body, 0, unroll=4)
    pltpu.sync_copy(z_v, agg_sh.at[pl.ds(sid * CR, CR)])

    @pl.when(sid == 0)
    def _():
        pltpu.sync_copy(z_v.at[pl.ds(0, CR_TAIL)],
                        agg_sh.at[pl.ds(NS * CR, CR_TAIL)])

    plsc.subcore_barrier()

    def start(c, slot):
        std, sts, se = sems[slot]
        pltpu.async_copy(td_hbm.at[idx_v.at[c, 1]], gtd_v.at[slot], std)
        pltpu.async_copy(ts_hbm.at[idx_v.at[c, 0]], gts_v.at[slot], sts)
        pltpu.async_copy(ep_hbm.at[pl.ds((wid * CPW + c) * CHUNK, CHUNK)],
                         ge_v.at[slot], se)

    def wait(slot):
        std, sts, se = sems[slot]
        pltpu.make_async_copy(td_hbm.at[idx_v.at[0, 1]], gtd_v.at[slot], std).wait()
        pltpu.make_async_copy(ts_hbm.at[idx_v.at[0, 0]], gts_v.at[slot], sts).wait()
        pltpu.make_async_copy(ep_hbm.at[pl.ds(0, CHUNK)], ge_v.at[slot], se).wait()

    def compute_scatter(c, slot):
        td, ts, e = gtd_v.at[slot], gts_v.at[slot], ge_v.at[slot]

        @plsc.parallel_loop(0, CHUNK, unroll=4)
        def _(i):
            f0 = td[i, pl.ds(0, 16)] + ts[i, pl.ds(0, 16)] + e[i, pl.ds(0, 16)]
            f1 = td[i, pl.ds(16, 16)] + ts[i, pl.ds(16, 16)] + e[i, pl.ds(16, 16)]
            s0 = td[i, pl.ds(32, 16)] + ts[i, pl.ds(32, 16)] + e[i, pl.ds(32, 16)]
            s1 = td[i, pl.ds(48, 16)] + ts[i, pl.ds(48, 16)] + e[i, pl.ds(48, 16)]
            m_v[i, pl.ds(0, 16)] = _gate(f0, s0)
            m_v[i, pl.ds(16, 16)] = _gate(f1, s1)

        pltpu.sync_copy(m_v, agg_sh.at[idx_v.at[c, 1]], add=True)

    start(0, 0)

    def pipe_body(jj, _):
        c0 = 2 * jj
        c1 = c0 + 1
        start(c1, 1)
        wait(0)
        compute_scatter(c0, 0)

        @pl.when(c1 + 1 < CPW)
        def _():
            start(c1 + 1, 0)

        wait(1)
        compute_scatter(c1, 1)
        return 0

    lax.fori_loop(0, CPW // 2, pipe_body, 0)
    plsc.subcore_barrier()

    pltpu.sync_copy(agg_sh.at[pl.ds(sid * CR, CR)], z_v)
    pltpu.sync_copy(z_v, out_hbm.at[cid, pl.ds(sid * CR, CR)])

    @pl.when(sid == 0)
    def _():
        pltpu.sync_copy(agg_sh.at[pl.ds(NS * CR, OUT_TAIL)],
                        z_v.at[pl.ds(0, OUT_TAIL)])
        pltpu.sync_copy(z_v.at[pl.ds(0, OUT_TAIL)],
                        out_hbm.at[cid, pl.ds(NS * CR, OUT_TAIL)])


def _edge_pass(td_pad, ts_pad, ep, idx):
    mesh = plsc.VectorSubcoreMesh(core_axis_name="c", subcore_axis_name="s",
                                  num_cores=NC, num_subcores=NS)
    f = functools.partial(
        pl.kernel,
        out_type=jax.ShapeDtypeStruct((NC, N, H), jnp.float32),
        mesh=mesh,
        scratch_types=[
            pltpu.VMEM((CPW, 2, CHUNK), jnp.int32),
            pltpu.VMEM((2, CHUNK, 2 * H), jnp.float32),
            pltpu.VMEM((2, CHUNK, 2 * H), jnp.float32),
            pltpu.VMEM((2, CHUNK, 2 * H), jnp.float32),
            pltpu.VMEM((CHUNK, H), jnp.float32),
            pltpu.VMEM((CR, H), jnp.float32),
            pltpu.VMEM_SHARED((N_AGG, H), jnp.float32),
            pltpu.SemaphoreType.DMA,
            pltpu.SemaphoreType.DMA,
            pltpu.SemaphoreType.DMA,
            pltpu.SemaphoreType.DMA,
            pltpu.SemaphoreType.DMA,
            pltpu.SemaphoreType.DMA,
        ],
        compiler_params=pltpu.CompilerParams(use_tc_tiling_on_sc=False),
    )(_edge_pass_body)
    return f(td_pad, ts_pad, ep, idx)


def _fold_bn(w, b, g, be, m, v, eps=1e-5):
    scale = g / jnp.sqrt(v + eps)
    return w * scale[None, :], (b - m) * scale + be


def kernel(x, edge_index, edge_attr, W_in, b_in, g_in, be_in, m_in, v_in,
           W_e, b_e, g_e, be_e, m_e, v_e, Wf1, bf1, Ws1, bs1, Wf2, bf2,
           Ws2, bs2, W_out, b_out):
    win, bin_ = _fold_bn(W_in, b_in, g_in, be_in, m_in, v_in)
    we, be = _fold_bn(W_e, b_e, g_e, be_e, m_e, v_e)

    wtd1 = jnp.concatenate([Wf1[0:H], Ws1[0:H]], axis=1)
    btd1 = jnp.concatenate([bf1, bs1])
    wts1 = jnp.concatenate([Wf1[H:2 * H], Ws1[H:2 * H]], axis=1)
    wtd2 = jnp.concatenate([Wf2[0:H], Ws2[0:H]], axis=1)
    btd2 = jnp.concatenate([bf2, bs2])
    wts2 = jnp.concatenate([Wf2[H:2 * H], Ws2[H:2 * H]], axis=1)
    wcat = jnp.concatenate([Wf1[2 * H:], Ws1[2 * H:], Wf2[2 * H:], Ws2[2 * H:]],
                           axis=1)

    src = edge_index[0]
    dst = edge_index[1]
    pad = E_PAD - E
    src_p = jnp.concatenate([src, jnp.zeros((pad,), jnp.int32)])
    dst_p = jnp.concatenate([dst, jnp.full((pad,), N, jnp.int32)])
    idx = jnp.stack([src_p, dst_p], 0).reshape(2, NCHUNKS, CHUNK).transpose(1, 0, 2)
    ea_pad = jnp.pad(edge_attr, ((0, pad), (0, 0)))

    ep1, ep2 = _edge_pre(ea_pad, we, be, wcat)

    h0, td1, ts1 = _node_pre(x, win, bin_, wtd1, btd1, wts1)

    td1p = jnp.pad(td1, ((0, N_TAB - N), (0, 0)))
    ts1p = jnp.pad(ts1, ((0, N_TAB - N), (0, 0)))
    agg1 = _edge_pass(td1p, ts1p, ep1, idx)

    h1, td2, ts2 = _node_mid(h0, agg1, wtd2, btd2, wts2)

    td2p = jnp.pad(td2, ((0, N_TAB - N), (0, 0)))
    ts2p = jnp.pad(ts2, ((0, N_TAB - N), (0, 0)))
    agg2 = _edge_pass(td2p, ts2p, ep2, idx)

    return _node_out(h1, agg2, W_out, b_out)


# speedup vs baseline: 4.6594x; 1.3290x over previous
import functools

import numpy as np

import jax
import jax.numpy as jnp
from jax import lax
from jax.experimental import pallas as pl
from jax.experimental.pallas import tpu as pltpu
from jax.experimental.pallas import tpu_sc as plsc

N, E, D, DE, H, O = 10000, 320000, 128, 16, 32, 128

NC, NS = 2, 16
NW = NC * NS
CHUNK = 128
CPW = 80
NCHUNKS = NW * CPW
E_PAD = NCHUNKS * CHUNK
N_TAB = N + 16
N_AGG = N + 16
CR = 624
CR_TAIL = N_AGG - NS * CR
OUT_TAIL = N - NS * CR

ILV = np.array([(k // 2) + 16 * (k % 2) for k in range(32)])

_LN_C = [
    -1.9367697179748704, 3.5140872970008568, -2.440029762615309,
    1.1160900268329503, -0.28382684778232653, 0.030449004538698962,
]
_LN_C[0] -= sum(_LN_C)


def _ln_poly(u):
    r = jnp.full_like(u, _LN_C[5])
    for c in (_LN_C[4], _LN_C[3], _LN_C[2], _LN_C[1], _LN_C[0]):
        r = r * u + c
    return r


def _gate(f, s):
    sig = 1.0 / (1.0 + jnp.exp(-f))
    sp = jnp.maximum(s, 0.0) + _ln_poly(1.0 + jnp.exp(-jnp.abs(s)))
    return sig * sp


def _leaky(h):
    return jnp.where(h >= 0, h, 0.1 * h)


def _edge_pre_body(ea_ref, we_ref, be_ref, wcat_ref, ep1_ref, ep2_ref):
    e = _leaky(jnp.dot(ea_ref[...], we_ref[...],
                       preferred_element_type=jnp.float32) + be_ref[...])
    r = jnp.dot(e, wcat_ref[...], preferred_element_type=jnp.float32)
    r = r.astype(jnp.bfloat16)
    ep1_ref[...] = r[:, :64]
    ep2_ref[...] = r[:, 64:]


def _edge_pre(ea_pad, we, be, wcat):
    BE = 4096
    grid = E_PAD // BE
    return pl.pallas_call(
        _edge_pre_body,
        grid=(grid,),
        in_specs=[
            pl.BlockSpec((BE, DE), lambda i: (i, 0)),
            pl.BlockSpec((DE, H), lambda i: (0, 0)),
            pl.BlockSpec((1, H), lambda i: (0, 0)),
            pl.BlockSpec((H, 128), lambda i: (0, 0)),
        ],
        out_specs=[
            pl.BlockSpec((BE, 2 * H), lambda i: (i, 0)),
            pl.BlockSpec((BE, 2 * H), lambda i: (i, 0)),
        ],
        out_shape=[
            jax.ShapeDtypeStruct((E_PAD, 2 * H), jnp.bfloat16),
            jax.ShapeDtypeStruct((E_PAD, 2 * H), jnp.bfloat16),
        ],
    )(ea_pad, we, be.reshape(1, H), wcat)


def _node_pre_body(x_ref, win_ref, bin_ref, wtd_ref, btd_ref, wts_ref,
                   h_ref, td_ref, ts_ref):
    h = _leaky(jnp.dot(x_ref[...], win_ref[...],
                       preferred_element_type=jnp.float32) + bin_ref[...])
    h_ref[...] = h
    td_ref[...] = (jnp.dot(h, wtd_ref[...], preferred_element_type=jnp.float32)
                   + btd_ref[...]).astype(jnp.bfloat16)
    ts_ref[...] = jnp.dot(
        h, wts_ref[...], preferred_element_type=jnp.float32
    ).astype(jnp.bfloat16)


def _node_pre(x, win, bin_, wtd, btd, wts):
    BN = 2000
    grid = N // BN
    return pl.pallas_call(
        _node_pre_body,
        grid=(grid,),
        in_specs=[
            pl.BlockSpec((BN, D), lambda i: (i, 0)),
            pl.BlockSpec((D, H), lambda i: (0, 0)),
            pl.BlockSpec((1, H), lambda i: (0, 0)),
            pl.BlockSpec((H, 2 * H), lambda i: (0, 0)),
            pl.BlockSpec((1, 2 * H), lambda i: (0, 0)),
            pl.BlockSpec((H, 2 * H), lambda i: (0, 0)),
        ],
        out_specs=[
            pl.BlockSpec((BN, H), lambda i: (i, 0)),
            pl.BlockSpec((BN, 2 * H), lambda i: (i, 0)),
            pl.BlockSpec((BN, 2 * H), lambda i: (i, 0)),
        ],
        out_shape=[
            jax.ShapeDtypeStruct((N, H), jnp.float32),
            jax.ShapeDtypeStruct((N, 2 * H), jnp.bfloat16),
            jax.ShapeDtypeStruct((N, 2 * H), jnp.bfloat16),
        ],
    )(x, win, bin_.reshape(1, H), wtd, btd.reshape(1, 2 * H), wts)


def _node_mid_body(h_ref, agg_ref, wtd_ref, btd_ref, wts_ref,
                   h1_ref, td_ref, ts_ref):
    h1 = h_ref[...] + agg_ref[0] + agg_ref[1]
    h1_ref[...] = h1
    td_ref[...] = (jnp.dot(h1, wtd_ref[...], preferred_element_type=jnp.float32)
                   + btd_ref[...]).astype(jnp.bfloat16)
    ts_ref[...] = jnp.dot(
        h1, wts_ref[...], preferred_element_type=jnp.float32
    ).astype(jnp.bfloat16)


def _node_mid(h, agg, wtd, btd, wts):
    BN = 2000
    grid = N // BN
    return pl.pallas_call(
        _node_mid_body,
        grid=(grid,),
        in_specs=[
            pl.BlockSpec((BN, H), lambda i: (i, 0)),
            pl.BlockSpec((NC, BN, H), lambda i: (0, i, 0)),
            pl.BlockSpec((H, 2 * H), lambda i: (0, 0)),
            pl.BlockSpec((1, 2 * H), lambda i: (0, 0)),
            pl.BlockSpec((H, 2 * H), lambda i: (0, 0)),
        ],
        out_specs=[
            pl.BlockSpec((BN, H), lambda i: (i, 0)),
            pl.BlockSpec((BN, 2 * H), lambda i: (i, 0)),
            pl.BlockSpec((BN, 2 * H), lambda i: (i, 0)),
        ],
        out_shape=[
            jax.ShapeDtypeStruct((N, H), jnp.float32),
            jax.ShapeDtypeStruct((N, 2 * H), jnp.bfloat16),
            jax.ShapeDtypeStruct((N, 2 * H), jnp.bfloat16),
        ],
    )(h, agg, wtd, btd.reshape(1, 2 * H), wts)


def _node_out_body(h_ref, agg_ref, wout_ref, bout_ref, out_ref):
    h2 = h_ref[...] + agg_ref[0] + agg_ref[1]
    out_ref[...] = jnp.dot(h2, wout_ref[...],
                           preferred_element_type=jnp.float32) + bout_ref[...]


def _node_out(h, agg, wout, bout):
    BN = 2000
    grid = N // BN
    return pl.pallas_call(
        _node_out_body,
        grid=(grid,),
        in_specs=[
            pl.BlockSpec((BN, H), lambda i: (i, 0)),
            pl.BlockSpec((NC, BN, H), lambda i: (0, i, 0)),
            pl.BlockSpec((H, O), lambda i: (0, 0)),
            pl.BlockSpec((1, O), lambda i: (0, 0)),
        ],
        out_specs=pl.BlockSpec((BN, O), lambda i: (i, 0)),
        out_shape=jax.ShapeDtypeStruct((N, O), jnp.float32),
    )(h, agg, wout, bout.reshape(1, O))


def _edge_pass_body(td_hbm, ts_hbm, ep_hbm, idx_hbm, out_hbm,
                    idx_v, gtd_v, gts_v, ge_v, m_v, z_v, agg_sh,
                    sem_td0, sem_td1, sem_ts0, sem_ts1, sem_e0, sem_e1):
    cid = lax.axis_index("c")
    sid = lax.axis_index("s")
    wid = sid * NC + cid
    sems = ((sem_td0, sem_ts0, sem_e0), (sem_td1, sem_ts1, sem_e1))

    pltpu.sync_copy(idx_hbm.at[pl.ds(wid * CPW, CPW)], idx_v)

    zero16 = jnp.zeros((16,), jnp.float32)

    def zero_body(i, _):
        z_v[i, pl.ds(0, 16)] = zero16
        z_v[i, pl.ds(16, 16)] = zero16
        return 0

    lax.fori_loop(0, CR, zero_body, 0, unroll=4)
    pltpu.sync_copy(z_v, agg_sh.at[pl.ds(sid * CR, CR)])

    @pl.when(sid == 0)
    def _():
        pltpu.sync_copy(z_v.at[pl.ds(0, CR_TAIL)],
                        agg_sh.at[pl.ds(NS * CR, CR_TAIL)])

    plsc.subcore_barrier()

    def start(c, slot):
        std, sts, se = sems[slot]
        pltpu.async_copy(td_hbm.at[idx_v.at[c, 1]], gtd_v.at[slot], std)
        pltpu.async_copy(ts_hbm.at[idx_v.at[c, 0]], gts_v.at[slot], sts)
        pltpu.async_copy(ep_hbm.at[pl.ds((wid * CPW + c) * CHUNK, CHUNK)],
                         ge_v.at[slot], se)

    def wait(slot):
        std, sts, se = sems[slot]
        pltpu.make_async_copy(td_hbm.at[idx_v.at[0, 1]], gtd_v.at[slot], std).wait()
        pltpu.make_async_copy(ts_hbm.at[idx_v.at[0, 0]], gts_v.at[slot], sts).wait()
        pltpu.make_async_copy(ep_hbm.at[pl.ds(0, CHUNK)], ge_v.at[slot], se).wait()

    def compute_scatter(c, slot):
        td, ts, e = gtd_v.at[slot], gts_v.at[slot], ge_v.at[slot]
        unp = functools.partial(plsc.unpack, format=plsc.PackFormat.INTERLEAVED)

        @plsc.parallel_loop(0, CHUNK, unroll=4)
        def _(i):
            fd0, fd1 = unp(td[i, pl.ds(0, 32)])
            sd0, sd1 = unp(td[i, pl.ds(32, 32)])
            fs0, fs1 = unp(ts[i, pl.ds(0, 32)])
            ss0, ss1 = unp(ts[i, pl.ds(32, 32)])
            fe0, fe1 = unp(e[i, pl.ds(0, 32)])
            se0, se1 = unp(e[i, pl.ds(32, 32)])
            m_v[i, pl.ds(0, 16)] = _gate(fd0 + fs0 + fe0, sd0 + ss0 + se0)
            m_v[i, pl.ds(16, 16)] = _gate(fd1 + fs1 + fe1, sd1 + ss1 + se1)

        pltpu.sync_copy(m_v, agg_sh.at[idx_v.at[c, 1]], add=True)

    start(0, 0)

    def pipe_body(jj, _):
        c0 = 2 * jj
        c1 = c0 + 1
        start(c1, 1)
        wait(0)
        compute_scatter(c0, 0)

        @pl.when(c1 + 1 < CPW)
        def _():
            start(c1 + 1, 0)

        wait(1)
        compute_scatter(c1, 1)
        return 0

    lax.fori_loop(0, CPW // 2, pipe_body, 0)
    plsc.subcore_barrier()

    pltpu.sync_copy(agg_sh.at[pl.ds(sid * CR, CR)], z_v)
    pltpu.sync_copy(z_v, out_hbm.at[cid, pl.ds(sid * CR, CR)])

    @pl.when(sid == 0)
    def _():
        pltpu.sync_copy(agg_sh.at[pl.ds(NS * CR, OUT_TAIL)],
                        z_v.at[pl.ds(0, OUT_TAIL)])
        pltpu.sync_copy(z_v.at[pl.ds(0, OUT_TAIL)],
                        out_hbm.at[cid, pl.ds(NS * CR, OUT_TAIL)])


def _edge_pass(td_pad, ts_pad, ep, idx):
    mesh = plsc.VectorSubcoreMesh(core_axis_name="c", subcore_axis_name="s",
                                  num_cores=NC, num_subcores=NS)
    f = functools.partial(
        pl.kernel,
        out_type=jax.ShapeDtypeStruct((NC, N, H), jnp.float32),
        mesh=mesh,
        scratch_types=[
            pltpu.VMEM((CPW, 2, CHUNK), jnp.int32),
            pltpu.VMEM((2, CHUNK, 2 * H), jnp.bfloat16),
            pltpu.VMEM((2, CHUNK, 2 * H), jnp.bfloat16),
            pltpu.VMEM((2, CHUNK, 2 * H), jnp.bfloat16),
            pltpu.VMEM((CHUNK, H), jnp.float32),
            pltpu.VMEM((CR, H), jnp.float32),
            pltpu.VMEM_SHARED((N_AGG, H), jnp.float32),
            pltpu.SemaphoreType.DMA,
            pltpu.SemaphoreType.DMA,
            pltpu.SemaphoreType.DMA,
            pltpu.SemaphoreType.DMA,
            pltpu.SemaphoreType.DMA,
            pltpu.SemaphoreType.DMA,
        ],
        compiler_params=pltpu.CompilerParams(use_tc_tiling_on_sc=False,
                                             needs_layout_passes=False),
    )(_edge_pass_body)
    return f(td_pad, ts_pad, ep, idx)


def _fold_bn(w, b, g, be, m, v, eps=1e-5):
    scale = g / jnp.sqrt(v + eps)
    return w * scale[None, :], (b - m) * scale + be


def kernel(x, edge_index, edge_attr, W_in, b_in, g_in, be_in, m_in, v_in,
           W_e, b_e, g_e, be_e, m_e, v_e, Wf1, bf1, Ws1, bs1, Wf2, bf2,
           Ws2, bs2, W_out, b_out):
    win, bin_ = _fold_bn(W_in, b_in, g_in, be_in, m_in, v_in)
    we, be = _fold_bn(W_e, b_e, g_e, be_e, m_e, v_e)

    wtd1 = jnp.concatenate([Wf1[0:H][:, ILV], Ws1[0:H][:, ILV]], axis=1)
    btd1 = jnp.concatenate([bf1[ILV], bs1[ILV]])
    wts1 = jnp.concatenate([Wf1[H:2 * H][:, ILV], Ws1[H:2 * H][:, ILV]], axis=1)
    wtd2 = jnp.concatenate([Wf2[0:H][:, ILV], Ws2[0:H][:, ILV]], axis=1)
    btd2 = jnp.concatenate([bf2[ILV], bs2[ILV]])
    wts2 = jnp.concatenate([Wf2[H:2 * H][:, ILV], Ws2[H:2 * H][:, ILV]], axis=1)
    wcat = jnp.concatenate([Wf1[2 * H:][:, ILV], Ws1[2 * H:][:, ILV],
                            Wf2[2 * H:][:, ILV], Ws2[2 * H:][:, ILV]], axis=1)

    src = edge_index[0]
    dst = edge_index[1]
    pad = E_PAD - E
    src_p = jnp.concatenate([src, jnp.zeros((pad,), jnp.int32)])
    dst_p = jnp.concatenate([dst, jnp.full((pad,), N, jnp.int32)])
    idx = jnp.stack([src_p, dst_p], 0).reshape(2, NCHUNKS, CHUNK).transpose(1, 0, 2)
    ea_pad = jnp.pad(edge_attr, ((0, pad), (0, 0)))

    ep1, ep2 = _edge_pre(ea_pad, we, be, wcat)

    h0, td1, ts1 = _node_pre(x, win, bin_, wtd1, btd1, wts1)

    td1p = jnp.pad(td1, ((0, N_TAB - N), (0, 0)))
    ts1p = jnp.pad(ts1, ((0, N_TAB - N), (0, 0)))
    agg1 = _edge_pass(td1p, ts1p, ep1, idx)

    h1, td2, ts2 = _node_mid(h0, agg1, wtd2, btd2, wts2)

    td2p = jnp.pad(td2, ((0, N_TAB - N), (0, 0)))
    ts2p = jnp.pad(ts2, ((0, N_TAB - N), (0, 0)))
    agg2 = _edge_pass(td2p, ts2p, ep2, idx)

    return _node_out(h1, agg2, W_out, b_out)


# speedup vs baseline: 4.9864x; 1.0702x over previous
import functools

import jax
import jax.numpy as jnp
from jax import lax
from jax.experimental import pallas as pl
from jax.experimental.pallas import tpu as pltpu
from jax.experimental.pallas import tpu_sc as plsc

N, E, D, DE, H, O = 10000, 320000, 128, 16, 32, 128

NC, NS = 2, 16
NW = NC * NS
CHUNK = 128
CPW = 80
NCHUNKS = NW * CPW
E_PAD = NCHUNKS * CHUNK
N_TAB = N + 16
N_AGG = N + 16
CR = 624
CR_TAIL = N_AGG - NS * CR
OUT_TAIL = N - NS * CR

_LN_C = [
    -1.9367697179748704, 3.5140872970008568, -2.440029762615309,
    1.1160900268329503, -0.28382684778232653, 0.030449004538698962,
]
_LN_C[0] -= sum(_LN_C)


def _ln_poly(u):
    r = jnp.full_like(u, _LN_C[5])
    for c in (_LN_C[4], _LN_C[3], _LN_C[2], _LN_C[1], _LN_C[0]):
        r = r * u + c
    return r


def _gate(f, s):
    sig = 1.0 / (1.0 + jnp.exp(-f))
    sp = jnp.maximum(s, 0.0) + _ln_poly(1.0 + jnp.exp(-jnp.abs(s)))
    return sig * sp


def _leaky(h):
    return jnp.where(h >= 0, h, 0.1 * h)


def _pack_words(r):
    outs = []
    for g in range(r.shape[1] // 32):
        blk = r[:, g * 32:(g + 1) * 32]
        lo = blk[:, :16].astype(jnp.bfloat16)
        hi = blk[:, 16:].astype(jnp.bfloat16)
        lo32 = lax.bitcast_convert_type(lo, jnp.uint16).astype(jnp.uint32)
        hi32 = lax.bitcast_convert_type(hi, jnp.uint16).astype(jnp.uint32)
        outs.append(lo32 | (hi32 << 16))
    return jnp.concatenate(outs, axis=1)


def _edge_pre_body(ea_ref, we_ref, be_ref, wcat_ref, ep1_ref, ep2_ref):
    e = _leaky(jnp.dot(ea_ref[...], we_ref[...],
                       preferred_element_type=jnp.float32) + be_ref[...])
    r = jnp.dot(e, wcat_ref[...], preferred_element_type=jnp.float32)
    ep1_ref[...] = _pack_words(r[:, :64])
    ep2_ref[...] = _pack_words(r[:, 64:])


def _edge_pre(edge_attr, we, be, wcat):
    BE = 4000
    grid = E // BE
    return pl.pallas_call(
        _edge_pre_body,
        grid=(grid,),
        in_specs=[
            pl.BlockSpec((BE, DE), lambda i: (i, 0)),
            pl.BlockSpec((DE, H), lambda i: (0, 0)),
            pl.BlockSpec((1, H), lambda i: (0, 0)),
            pl.BlockSpec((H, 128), lambda i: (0, 0)),
        ],
        out_specs=[
            pl.BlockSpec((BE, H), lambda i: (i, 0)),
            pl.BlockSpec((BE, H), lambda i: (i, 0)),
        ],
        out_shape=[
            jax.ShapeDtypeStruct((E_PAD, H), jnp.uint32),
            jax.ShapeDtypeStruct((E_PAD, H), jnp.uint32),
        ],
    )(edge_attr, we, be.reshape(1, H), wcat)


def _node_pre_body(x_ref, win_ref, bin_ref, wt_ref, bt_ref,
                   h_ref, td_ref, ts_ref):
    h = _leaky(jnp.dot(x_ref[...], win_ref[...],
                       preferred_element_type=jnp.float32) + bin_ref[...])
    h_ref[...] = h
    t = jnp.dot(h, wt_ref[...], preferred_element_type=jnp.float32) + bt_ref[...]
    td_ref[...] = _pack_words(t[:, :64])
    ts_ref[...] = _pack_words(t[:, 64:])


def _node_pre(x, win, bin_, wt, bt):
    BN = 2000
    grid = N // BN
    return pl.pallas_call(
        _node_pre_body,
        grid=(grid,),
        in_specs=[
            pl.BlockSpec((BN, D), lambda i: (i, 0)),
            pl.BlockSpec((D, H), lambda i: (0, 0)),
            pl.BlockSpec((1, H), lambda i: (0, 0)),
            pl.BlockSpec((H, 4 * H), lambda i: (0, 0)),
            pl.BlockSpec((1, 4 * H), lambda i: (0, 0)),
        ],
        out_specs=[
            pl.BlockSpec((BN, H), lambda i: (i, 0)),
            pl.BlockSpec((BN, H), lambda i: (i, 0)),
            pl.BlockSpec((BN, H), lambda i: (i, 0)),
        ],
        out_shape=[
            jax.ShapeDtypeStruct((N, H), jnp.float32),
            jax.ShapeDtypeStruct((N, H), jnp.uint32),
            jax.ShapeDtypeStruct((N, H), jnp.uint32),
        ],
    )(x, win, bin_.reshape(1, H), wt, bt.reshape(1, 4 * H))


def _node_mid_body(h_ref, agg_ref, wt_ref, bt_ref, h1_ref, td_ref, ts_ref):
    h1 = h_ref[...] + agg_ref[0] + agg_ref[1]
    h1_ref[...] = h1
    t = jnp.dot(h1, wt_ref[...], preferred_element_type=jnp.float32) + bt_ref[...]
    td_ref[...] = _pack_words(t[:, :64])
    ts_ref[...] = _pack_words(t[:, 64:])


def _node_mid(h, agg, wt, bt):
    BN = 2000
    grid = N // BN
    return pl.pallas_call(
        _node_mid_body,
        grid=(grid,),
        in_specs=[
            pl.BlockSpec((BN, H), lambda i: (i, 0)),
            pl.BlockSpec((NC, BN, H), lambda i: (0, i, 0)),
            pl.BlockSpec((H, 4 * H), lambda i: (0, 0)),
            pl.BlockSpec((1, 4 * H), lambda i: (0, 0)),
        ],
        out_specs=[
            pl.BlockSpec((BN, H), lambda i: (i, 0)),
            pl.BlockSpec((BN, H), lambda i: (i, 0)),
            pl.BlockSpec((BN, H), lambda i: (i, 0)),
        ],
        out_shape=[
            jax.ShapeDtypeStruct((N, H), jnp.float32),
            jax.ShapeDtypeStruct((N, H), jnp.uint32),
            jax.ShapeDtypeStruct((N, H), jnp.uint32),
        ],
    )(h, agg, wt, bt.reshape(1, 4 * H))


def _node_out_body(h_ref, agg_ref, wout_ref, bout_ref, out_ref):
    h2 = h_ref[...] + agg_ref[0] + agg_ref[1]
    out_ref[...] = jnp.dot(h2, wout_ref[...],
                           preferred_element_type=jnp.float32) + bout_ref[...]


def _node_out(h, agg, wout, bout):
    BN = 2000
    grid = N // BN
    return pl.pallas_call(
        _node_out_body,
        grid=(grid,),
        in_specs=[
            pl.BlockSpec((BN, H), lambda i: (i, 0)),
            pl.BlockSpec((NC, BN, H), lambda i: (0, i, 0)),
            pl.BlockSpec((H, O), lambda i: (0, 0)),
            pl.BlockSpec((1, O), lambda i: (0, 0)),
        ],
        out_specs=pl.BlockSpec((BN, O), lambda i: (i, 0)),
        out_shape=jax.ShapeDtypeStruct((N, O), jnp.float32),
    )(h, agg, wout, bout.reshape(1, O))


def _edge_pass_body(td_hbm, ts_hbm, ep_hbm, idx_hbm, out_hbm,
                    idx_v, gtd_v, gts_v, ge_v, m_v, z_v, agg_sh,
                    sem_td0, sem_td1, sem_ts0, sem_ts1, sem_e0, sem_e1):
    cid = lax.axis_index("c")
    sid = lax.axis_index("s")
    wid = sid * NC + cid
    sems = ((sem_td0, sem_ts0, sem_e0), (sem_td1, sem_ts1, sem_e1))

    pltpu.sync_copy(idx_hbm.at[0, pl.ds(wid * CPW, CPW)], idx_v.at[0])
    pltpu.sync_copy(idx_hbm.at[1, pl.ds(wid * CPW, CPW)], idx_v.at[1])

    zero16 = jnp.zeros((16,), jnp.float32)

    def zero_body(i, _):
        z_v[i, pl.ds(0, 16)] = zero16
        z_v[i, pl.ds(16, 16)] = zero16
        return 0

    lax.fori_loop(0, CR, zero_body, 0, unroll=4)
    pltpu.sync_copy(z_v, agg_sh.at[pl.ds(sid * CR, CR)])

    @pl.when(sid == 0)
    def _():
        pltpu.sync_copy(z_v.at[pl.ds(0, CR_TAIL)],
                        agg_sh.at[pl.ds(NS * CR, CR_TAIL)])

    plsc.subcore_barrier()

    def start(c, slot):
        std, sts, se = sems[slot]
        pltpu.async_copy(td_hbm.at[idx_v.at[1, c]], gtd_v.at[slot], std)
        pltpu.async_copy(ts_hbm.at[idx_v.at[0, c]], gts_v.at[slot], sts)
        pltpu.async_copy(ep_hbm.at[pl.ds((wid * CPW + c) * CHUNK, CHUNK)],
                         ge_v.at[slot], se)

    def wait(slot):
        std, sts, se = sems[slot]
        pltpu.make_async_copy(td_hbm.at[idx_v.at[1, 0]], gtd_v.at[slot],
                              std).wait()
        pltpu.make_async_copy(ts_hbm.at[idx_v.at[0, 0]], gts_v.at[slot],
                              sts).wait()
        pltpu.make_async_copy(ep_hbm.at[pl.ds(0, CHUNK)],
                              ge_v.at[slot], se).wait()

    def compute_scatter(c, slot):
        td, ts, e = gtd_v.at[slot], gts_v.at[slot], ge_v.at[slot]
        unp = functools.partial(plsc.unpack, format=plsc.PackFormat.INTERLEAVED)

        def unp16(ref, i, off):
            w = ref[i, pl.ds(off, 16)]
            return unp(plsc.bitcast(w, jnp.bfloat16))

        @plsc.parallel_loop(0, CHUNK, unroll=4)
        def _(i):
            fd0, fd1 = unp16(td, i, 0)
            sd0, sd1 = unp16(td, i, 16)
            fs0, fs1 = unp16(ts, i, 0)
            ss0, ss1 = unp16(ts, i, 16)
            fe0, fe1 = unp16(e, i, 0)
            se0, se1 = unp16(e, i, 16)
            m_v[i, pl.ds(0, 16)] = _gate(fd0 + fs0 + fe0, sd0 + ss0 + se0)
            m_v[i, pl.ds(16, 16)] = _gate(fd1 + fs1 + fe1, sd1 + ss1 + se1)

        pltpu.sync_copy(m_v, agg_sh.at[idx_v.at[1, c]], add=True)

    start(0, 0)

    def pipe_body(jj, _):
        c0 = 2 * jj
        c1 = c0 + 1
        start(c1, 1)
        wait(0)
        compute_scatter(c0, 0)

        @pl.when(c1 + 1 < CPW)
        def _():
            start(c1 + 1, 0)

        wait(1)
        compute_scatter(c1, 1)
        return 0

    lax.fori_loop(0, CPW // 2, pipe_body, 0)
    plsc.subcore_barrier()

    pltpu.sync_copy(agg_sh.at[pl.ds(sid * CR, CR)], z_v)
    pltpu.sync_copy(z_v, out_hbm.at[cid, pl.ds(sid * CR, CR)])

    @pl.when(sid == 0)
    def _():
        pltpu.sync_copy(agg_sh.at[pl.ds(NS * CR, OUT_TAIL)],
                        z_v.at[pl.ds(0, OUT_TAIL)])
        pltpu.sync_copy(z_v.at[pl.ds(0, OUT_TAIL)],
                        out_hbm.at[cid, pl.ds(NS * CR, OUT_TAIL)])


def _edge_pass(td_pad, ts_pad, ep, idx):
    mesh = plsc.VectorSubcoreMesh(core_axis_name="c", subcore_axis_name="s",
                                  num_cores=NC, num_subcores=NS)
    f = functools.partial(
        pl.kernel,
        out_type=jax.ShapeDtypeStruct((NC, N, H), jnp.float32),
        mesh=mesh,
        scratch_types=[
            pltpu.VMEM((2, CPW, CHUNK), jnp.int32),
            pltpu.VMEM((2, CHUNK, H), jnp.uint32),
            pltpu.VMEM((2, CHUNK, H), jnp.uint32),
            pltpu.VMEM((2, CHUNK, H), jnp.uint32),
            pltpu.VMEM((CHUNK, H), jnp.float32),
            pltpu.VMEM((CR, H), jnp.float32),
            pltpu.VMEM_SHARED((N_AGG, H), jnp.float32),
            pltpu.SemaphoreType.DMA,
            pltpu.SemaphoreType.DMA,
            pltpu.SemaphoreType.DMA,
            pltpu.SemaphoreType.DMA,
            pltpu.SemaphoreType.DMA,
            pltpu.SemaphoreType.DMA,
        ],
        compiler_params=pltpu.CompilerParams(use_tc_tiling_on_sc=False,
                                             needs_layout_passes=False),
    )(_edge_pass_body)
    return f(td_pad, ts_pad, ep, idx)


def _fold_bn(w, b, g, be, m, v, eps=1e-5):
    scale = g / jnp.sqrt(v + eps)
    return w * scale[None, :], (b - m) * scale + be


def kernel(x, edge_index, edge_attr, W_in, b_in, g_in, be_in, m_in, v_in,
           W_e, b_e, g_e, be_e, m_e, v_e, Wf1, bf1, Ws1, bs1, Wf2, bf2,
           Ws2, bs2, W_out, b_out):
    win, bin_ = _fold_bn(W_in, b_in, g_in, be_in, m_in, v_in)
    we, be = _fold_bn(W_e, b_e, g_e, be_e, m_e, v_e)

    zeros2h = jnp.zeros((2 * H,), jnp.float32)
    wt1 = jnp.concatenate([Wf1[0:H], Ws1[0:H], Wf1[H:2 * H], Ws1[H:2 * H]],
                          axis=1)
    bt1 = jnp.concatenate([bf1, bs1, zeros2h])
    wt2 = jnp.concatenate([Wf2[0:H], Ws2[0:H], Wf2[H:2 * H], Ws2[H:2 * H]],
                          axis=1)
    bt2 = jnp.concatenate([bf2, bs2, zeros2h])
    wcat = jnp.concatenate([Wf1[2 * H:], Ws1[2 * H:], Wf2[2 * H:], Ws2[2 * H:]],
                           axis=1)

    src = edge_index[0]
    dst = edge_index[1]
    pad = E_PAD - E
    src_p = jnp.concatenate([src, jnp.zeros((pad,), jnp.int32)])
    dst_p = jnp.concatenate([dst, jnp.full((pad,), N, jnp.int32)])
    idx = jnp.concatenate([src_p, dst_p]).reshape(2, NCHUNKS, CHUNK)

    ep1, ep2 = _edge_pre(edge_attr, we, be, wcat)

    h0, td1, ts1 = _node_pre(x, win, bin_, wt1, bt1)

    pad_t = ((0, N_TAB - N), (0, 0))
    agg1 = _edge_pass(jnp.pad(td1, pad_t), jnp.pad(ts1, pad_t), ep1, idx)

    h1, td2, ts2 = _node_mid(h0, agg1, wt2, bt2)

    agg2 = _edge_pass(jnp.pad(td2, pad_t), jnp.pad(ts2, pad_t), ep2, idx)

    return _node_out(h1, agg2, W_out, b_out)


# speedup vs baseline: 6.9515x; 1.3941x over previous
import functools

import jax
import jax.numpy as jnp
from jax import lax
from jax.experimental import pallas as pl
from jax.experimental.pallas import tpu as pltpu
from jax.experimental.pallas import tpu_sc as plsc

N, E, D, DE, H, O = 10000, 320000, 128, 16, 32, 128

NC, NS = 2, 16
NW = NC * NS
CHUNK = 128
CPW = 80
NCHUNKS = NW * CPW
E_PAD = NCHUNKS * CHUNK
N_TAB = N + 16
N_AGG = N + 16
CR = 624
CR_TAIL = N_AGG - NS * CR
OUT_TAIL = N - NS * CR

_LN_C = [
    -1.9367697179748704, 3.5140872970008568, -2.440029762615309,
    1.1160900268329503, -0.28382684778232653, 0.030449004538698962,
]
_LN_C[0] -= sum(_LN_C)


def _ln_poly(u):
    r = jnp.full_like(u, _LN_C[5])
    for c in (_LN_C[4], _LN_C[3], _LN_C[2], _LN_C[1], _LN_C[0]):
        r = r * u + c
    return r


def _gate(f, s):
    sig = 1.0 / (1.0 + jnp.exp(-f))
    sp = jnp.maximum(s, 0.0) + _ln_poly(1.0 + jnp.exp(-jnp.abs(s)))
    return sig * sp


def _leaky(h):
    return jnp.where(h >= 0, h, 0.1 * h)


def _pack_full(r):
    k = r.shape[1] // 2
    lo = r[:, :k].astype(jnp.bfloat16)
    hi = r[:, k:].astype(jnp.bfloat16)
    lo32 = lax.bitcast_convert_type(lo, jnp.uint16).astype(jnp.uint32)
    hi32 = lax.bitcast_convert_type(hi, jnp.uint16).astype(jnp.uint32)
    return lo32 | (hi32 << 16)


def _edge_pre_body(ea0, ea1, ea2, ea3, we_ref, be_ref, wcat_ref,
                   ep1_ref, ep2_ref):
    for q, ea_ref in enumerate((ea0, ea1, ea2, ea3)):
        e = _leaky(jnp.dot(ea_ref[...], we_ref[...],
                           preferred_element_type=jnp.float32) + be_ref[...])
        r = jnp.dot(e, wcat_ref[...], preferred_element_type=jnp.float32)
        w = _pack_full(r)
        ep1_ref[:, 32 * q:32 * q + 32] = w[:, :32]
        ep2_ref[:, 32 * q:32 * q + 32] = w[:, 32:]


def _edge_pre(edge_attr, we, be, wcat):
    BE = 2560
    QB = E_PAD // 4 // BE
    last = E // BE - 1

    def ea_map(q):
        return lambda i: (jnp.minimum(q * QB + i, last), 0)

    return pl.pallas_call(
        _edge_pre_body,
        grid=(QB,),
        in_specs=[
            pl.BlockSpec((BE, DE), ea_map(0)),
            pl.BlockSpec((BE, DE), ea_map(1)),
            pl.BlockSpec((BE, DE), ea_map(2)),
            pl.BlockSpec((BE, DE), ea_map(3)),
            pl.BlockSpec((DE, H), lambda i: (0, 0)),
            pl.BlockSpec((1, H), lambda i: (0, 0)),
            pl.BlockSpec((H, 128), lambda i: (0, 0)),
        ],
        out_specs=[
            pl.BlockSpec((BE, 128), lambda i: (i, 0)),
            pl.BlockSpec((BE, 128), lambda i: (i, 0)),
        ],
        out_shape=[
            jax.ShapeDtypeStruct((E_PAD // 4, 128), jnp.uint32),
            jax.ShapeDtypeStruct((E_PAD // 4, 128), jnp.uint32),
        ],
    )(edge_attr, edge_attr, edge_attr, edge_attr, we, be.reshape(1, H), wcat)


def _node_pre_body(x_ref, win_ref, bin_ref, wt_ref, bt_ref,
                   h_ref, td_ref, ts_ref):
    h = _leaky(jnp.dot(x_ref[...], win_ref[...],
                       preferred_element_type=jnp.float32) + bin_ref[...])
    h_ref[...] = h
    t = jnp.dot(h, wt_ref[...], preferred_element_type=jnp.float32) + bt_ref[...]
    w = _pack_full(t)
    td_ref[...] = w[:, :32]
    ts_ref[...] = w[:, 32:]


def _node_pre(x, win, bin_, wt, bt):
    BN = 2000
    grid = N // BN
    return pl.pallas_call(
        _node_pre_body,
        grid=(grid,),
        in_specs=[
            pl.BlockSpec((BN, D), lambda i: (i, 0)),
            pl.BlockSpec((D, H), lambda i: (0, 0)),
            pl.BlockSpec((1, H), lambda i: (0, 0)),
            pl.BlockSpec((H, 4 * H), lambda i: (0, 0)),
            pl.BlockSpec((1, 4 * H), lambda i: (0, 0)),
        ],
        out_specs=[
            pl.BlockSpec((BN, H), lambda i: (i, 0)),
            pl.BlockSpec((BN, H), lambda i: (i, 0)),
            pl.BlockSpec((BN, H), lambda i: (i, 0)),
        ],
        out_shape=[
            jax.ShapeDtypeStruct((N, H), jnp.float32),
            jax.ShapeDtypeStruct((N, H), jnp.uint32),
            jax.ShapeDtypeStruct((N, H), jnp.uint32),
        ],
    )(x, win, bin_.reshape(1, H), wt, bt.reshape(1, 4 * H))


def _node_mid_body(h_ref, agg_ref, wt_ref, bt_ref, h1_ref, td_ref, ts_ref):
    h1 = h_ref[...] + agg_ref[0] + agg_ref[1]
    h1_ref[...] = h1
    t = jnp.dot(h1, wt_ref[...], preferred_element_type=jnp.float32) + bt_ref[...]
    w = _pack_full(t)
    td_ref[...] = w[:, :32]
    ts_ref[...] = w[:, 32:]


def _node_mid(h, agg, wt, bt):
    BN = 2000
    grid = N // BN
    return pl.pallas_call(
        _node_mid_body,
        grid=(grid,),
        in_specs=[
            pl.BlockSpec((BN, H), lambda i: (i, 0)),
            pl.BlockSpec((NC, BN, H), lambda i: (0, i, 0)),
            pl.BlockSpec((H, 4 * H), lambda i: (0, 0)),
            pl.BlockSpec((1, 4 * H), lambda i: (0, 0)),
        ],
        out_specs=[
            pl.BlockSpec((BN, H), lambda i: (i, 0)),
            pl.BlockSpec((BN, H), lambda i: (i, 0)),
            pl.BlockSpec((BN, H), lambda i: (i, 0)),
        ],
        out_shape=[
            jax.ShapeDtypeStruct((N, H), jnp.float32),
            jax.ShapeDtypeStruct((N, H), jnp.uint32),
            jax.ShapeDtypeStruct((N, H), jnp.uint32),
        ],
    )(h, agg, wt, bt.reshape(1, 4 * H))


def _node_out_body(h_ref, agg_ref, wout_ref, bout_ref, out_ref):
    h2 = h_ref[...] + agg_ref[0] + agg_ref[1]
    out_ref[...] = jnp.dot(h2, wout_ref[...],
                           preferred_element_type=jnp.float32) + bout_ref[...]


def _node_out(h, agg, wout, bout):
    BN = 2000
    grid = N // BN
    return pl.pallas_call(
        _node_out_body,
        grid=(grid,),
        in_specs=[
            pl.BlockSpec((BN, H), lambda i: (i, 0)),
            pl.BlockSpec((NC, BN, H), lambda i: (0, i, 0)),
            pl.BlockSpec((H, O), lambda i: (0, 0)),
            pl.BlockSpec((1, O), lambda i: (0, 0)),
        ],
        out_specs=pl.BlockSpec((BN, O), lambda i: (i, 0)),
        out_shape=jax.ShapeDtypeStruct((N, O), jnp.float32),
    )(h, agg, wout, bout.reshape(1, O))


def _edge_pass_body(td_hbm, ts_hbm, ep_hbm, idx_hbm, out_hbm,
                    idx_v, gtd_v, gts_v, ge_v, m_v, z_v, agg_sh,
                    sem_td0, sem_td1, sem_ts0, sem_ts1, sem_e0, sem_e1):
    cid = lax.axis_index("c")
    sid = lax.axis_index("s")
    wid = sid * NC + cid
    epq = wid // 8
    epr = (wid % 8) * CPW
    sems = ((sem_td0, sem_ts0, sem_e0), (sem_td1, sem_ts1, sem_e1))

    pltpu.sync_copy(idx_hbm.at[0, pl.ds(wid * CPW, CPW)], idx_v.at[0])
    pltpu.sync_copy(idx_hbm.at[1, pl.ds(wid * CPW, CPW)], idx_v.at[1])

    zero16 = jnp.zeros((16,), jnp.float32)

    def zero_body(i, _):
        z_v[i, pl.ds(0, 16)] = zero16
        z_v[i, pl.ds(16, 16)] = zero16
        return 0

    lax.fori_loop(0, CR, zero_body, 0, unroll=4)
    pltpu.sync_copy(z_v, agg_sh.at[pl.ds(sid * CR, CR)])

    @pl.when(sid == 0)
    def _():
        pltpu.sync_copy(z_v.at[pl.ds(0, CR_TAIL)],
                        agg_sh.at[pl.ds(NS * CR, CR_TAIL)])

    plsc.subcore_barrier()

    def start(c, slot):
        std, sts, se = sems[slot]
        pltpu.async_copy(td_hbm.at[idx_v.at[1, c]], gtd_v.at[slot], std)
        pltpu.async_copy(ts_hbm.at[idx_v.at[0, c]], gts_v.at[slot], sts)
        pltpu.async_copy(ep_hbm.at[pl.ds((epr + c) * CHUNK, CHUNK),
                                   pl.ds(32 * epq, 32)], ge_v.at[slot], se)

    def wait(slot):
        std, sts, se = sems[slot]
        pltpu.make_async_copy(td_hbm.at[idx_v.at[1, 0]], gtd_v.at[slot],
                              std).wait()
        pltpu.make_async_copy(ts_hbm.at[idx_v.at[0, 0]], gts_v.at[slot],
                              sts).wait()
        pltpu.make_async_copy(ep_hbm.at[pl.ds(0, CHUNK), pl.ds(0, 32)],
                              ge_v.at[slot], se).wait()

    def compute_scatter(c, slot):
        td, ts, e = gtd_v.at[slot], gts_v.at[slot], ge_v.at[slot]
        unp = functools.partial(plsc.unpack, format=plsc.PackFormat.INTERLEAVED)

        def unp16(ref, i, off):
            w = ref[i, pl.ds(off, 16)]
            return unp(plsc.bitcast(w, jnp.bfloat16))

        @plsc.parallel_loop(0, CHUNK, unroll=4)
        def _(i):
            fd0, fd1 = unp16(td, i, 0)
            sd0, sd1 = unp16(td, i, 16)
            fs0, fs1 = unp16(ts, i, 0)
            ss0, ss1 = unp16(ts, i, 16)
            fe0, fe1 = unp16(e, i, 0)
            se0, se1 = unp16(e, i, 16)
            m_v[i, pl.ds(0, 16)] = _gate(fd0 + fs0 + fe0, sd0 + ss0 + se0)
            m_v[i, pl.ds(16, 16)] = _gate(fd1 + fs1 + fe1, sd1 + ss1 + se1)

        pltpu.sync_copy(m_v, agg_sh.at[idx_v.at[1, c]], add=True)

    start(0, 0)

    def pipe_body(jj, _):
        c0 = 2 * jj
        c1 = c0 + 1
        start(c1, 1)
        wait(0)
        compute_scatter(c0, 0)

        @pl.when(c1 + 1 < CPW)
        def _():
            start(c1 + 1, 0)

        wait(1)
        compute_scatter(c1, 1)
        return 0

    lax.fori_loop(0, CPW // 2, pipe_body, 0)
    plsc.subcore_barrier()

    pltpu.sync_copy(agg_sh.at[pl.ds(sid * CR, CR)], z_v)
    pltpu.sync_copy(z_v, out_hbm.at[cid, pl.ds(sid * CR, CR)])

    @pl.when(sid == 0)
    def _():
        pltpu.sync_copy(agg_sh.at[pl.ds(NS * CR, OUT_TAIL)],
                        z_v.at[pl.ds(0, OUT_TAIL)])
        pltpu.sync_copy(z_v.at[pl.ds(0, OUT_TAIL)],
                        out_hbm.at[cid, pl.ds(NS * CR, OUT_TAIL)])


def _edge_pass(td_pad, ts_pad, ep, idx):
    mesh = plsc.VectorSubcoreMesh(core_axis_name="c", subcore_axis_name="s",
                                  num_cores=NC, num_subcores=NS)
    f = functools.partial(
        pl.kernel,
        out_type=jax.ShapeDtypeStruct((NC, N, H), jnp.float32),
        mesh=mesh,
        scratch_types=[
            pltpu.VMEM((2, CPW, CHUNK), jnp.int32),
            pltpu.VMEM((2, CHUNK, H), jnp.uint32),
            pltpu.VMEM((2, CHUNK, H), jnp.uint32),
            pltpu.VMEM((2, CHUNK, H), jnp.uint32),
            pltpu.VMEM((CHUNK, H), jnp.float32),
            pltpu.VMEM((CR, H), jnp.float32),
            pltpu.VMEM_SHARED((N_AGG, H), jnp.float32),
            pltpu.SemaphoreType.DMA,
            pltpu.SemaphoreType.DMA,
            pltpu.SemaphoreType.DMA,
            pltpu.SemaphoreType.DMA,
            pltpu.SemaphoreType.DMA,
            pltpu.SemaphoreType.DMA,
        ],
        compiler_params=pltpu.CompilerParams(use_tc_tiling_on_sc=False,
                                             needs_layout_passes=False),
    )(_edge_pass_body)
    return f(td_pad, ts_pad, ep, idx)


def _fold_bn(w, b, g, be, m, v, eps=1e-5):
    scale = g / jnp.sqrt(v + eps)
    return w * scale[None, :], (b - m) * scale + be


def kernel(x, edge_index, edge_attr, W_in, b_in, g_in, be_in, m_in, v_in,
           W_e, b_e, g_e, be_e, m_e, v_e, Wf1, bf1, Ws1, bs1, Wf2, bf2,
           Ws2, bs2, W_out, b_out):
    win, bin_ = _fold_bn(W_in, b_in, g_in, be_in, m_in, v_in)
    we, be = _fold_bn(W_e, b_e, g_e, be_e, m_e, v_e)

    def _lohi(*blocks):
        return jnp.concatenate([b[:, :16] for b in blocks]
                               + [b[:, 16:] for b in blocks], axis=1)

    def _lohi_b(*vecs):
        return jnp.concatenate([v[:16] for v in vecs]
                               + [v[16:] for v in vecs])

    z32 = jnp.zeros((H,), jnp.float32)
    wt1 = _lohi(Wf1[0:H], Ws1[0:H], Wf1[H:2 * H], Ws1[H:2 * H])
    bt1 = _lohi_b(bf1, bs1, z32, z32)
    wt2 = _lohi(Wf2[0:H], Ws2[0:H], Wf2[H:2 * H], Ws2[H:2 * H])
    bt2 = _lohi_b(bf2, bs2, z32, z32)
    wcat = _lohi(Wf1[2 * H:], Ws1[2 * H:], Wf2[2 * H:], Ws2[2 * H:])

    src = edge_index[0]
    dst = edge_index[1]
    pad = E_PAD - E
    src_p = jnp.concatenate([src, jnp.zeros((pad,), jnp.int32)])
    dst_p = jnp.concatenate([dst, jnp.full((pad,), N, jnp.int32)])
    idx = jnp.concatenate([src_p, dst_p]).reshape(2, NCHUNKS, CHUNK)

    ep1, ep2 = _edge_pre(edge_attr, we, be, wcat)

    h0, td1, ts1 = _node_pre(x, win, bin_, wt1, bt1)

    pad_t = ((0, N_TAB - N), (0, 0))
    agg1 = _edge_pass(jnp.pad(td1, pad_t), jnp.pad(ts1, pad_t), ep1, idx)

    h1, td2, ts2 = _node_mid(h0, agg1, wt2, bt2)

    agg2 = _edge_pass(jnp.pad(td2, pad_t), jnp.pad(ts2, pad_t), ep2, idx)

    return _node_out(h1, agg2, W_out, b_out)


# speedup vs baseline: 7.1386x; 1.0269x over previous
import functools

import jax
import jax.numpy as jnp
from jax import lax
from jax.experimental import pallas as pl
from jax.experimental.pallas import tpu as pltpu
from jax.experimental.pallas import tpu_sc as plsc

N, E, D, DE, H, O = 10000, 320000, 128, 16, 32, 128

NC, NS = 2, 16
NW = NC * NS
CHUNK = 128
CPW = 80
NCHUNKS = NW * CPW
E_PAD = NCHUNKS * CHUNK
N_TAB = N + 16
N_AGG = N + 16
CR = 624
CR_TAIL = N_AGG - NS * CR
OUT_TAIL = N - NS * CR

_LN_C = [
    -1.9367697179748704, 3.5140872970008568, -2.440029762615309,
    1.1160900268329503, -0.28382684778232653, 0.030449004538698962,
]
_LN_C[0] -= sum(_LN_C)


def _ln_poly(u):
    r = jnp.full_like(u, _LN_C[5])
    for c in (_LN_C[4], _LN_C[3], _LN_C[2], _LN_C[1], _LN_C[0]):
        r = r * u + c
    return r


def _gate(f, s):
    sig = 1.0 / (1.0 + jnp.exp(-f))
    sp = jnp.maximum(s, 0.0) + _ln_poly(1.0 + jnp.exp(-jnp.abs(s)))
    return sig * sp


def _leaky(h):
    return jnp.where(h >= 0, h, 0.1 * h)


def _pack_full(r):
    k = r.shape[1] // 2
    lo = r[:, :k].astype(jnp.bfloat16)
    hi = r[:, k:].astype(jnp.bfloat16)
    lo32 = lax.bitcast_convert_type(lo, jnp.uint16).astype(jnp.uint32)
    hi32 = lax.bitcast_convert_type(hi, jnp.uint16).astype(jnp.uint32)
    return lo32 | (hi32 << 16)


def _edge_pre_body(ea0, ea1, ea2, ea3, we_ref, be_ref, wcat_ref,
                   ep1_ref, ep2_ref):
    for q, ea_ref in enumerate((ea0, ea1, ea2, ea3)):
        e = _leaky(jnp.dot(ea_ref[...], we_ref[...],
                           preferred_element_type=jnp.float32) + be_ref[...])
        r = jnp.dot(e, wcat_ref[...], preferred_element_type=jnp.float32)
        w = _pack_full(r)
        ep1_ref[:, 32 * q:32 * q + 32] = w[:, :32]
        ep2_ref[:, 32 * q:32 * q + 32] = w[:, 32:]


def _edge_pre(edge_attr, we, be, wcat):
    BE = 2560
    QB = E_PAD // 4 // BE
    last = E // BE - 1

    def ea_map(q):
        return lambda i: (jnp.minimum(q * QB + i, last), 0)

    return pl.pallas_call(
        _edge_pre_body,
        grid=(QB,),
        in_specs=[
            pl.BlockSpec((BE, DE), ea_map(0)),
            pl.BlockSpec((BE, DE), ea_map(1)),
            pl.BlockSpec((BE, DE), ea_map(2)),
            pl.BlockSpec((BE, DE), ea_map(3)),
            pl.BlockSpec((DE, H), lambda i: (0, 0)),
            pl.BlockSpec((1, H), lambda i: (0, 0)),
            pl.BlockSpec((H, 128), lambda i: (0, 0)),
        ],
        out_specs=[
            pl.BlockSpec((BE, 128), lambda i: (i, 0)),
            pl.BlockSpec((BE, 128), lambda i: (i, 0)),
        ],
        out_shape=[
            jax.ShapeDtypeStruct((E_PAD // 4, 128), jnp.uint32),
            jax.ShapeDtypeStruct((E_PAD // 4, 128), jnp.uint32),
        ],
    )(edge_attr, edge_attr, edge_attr, edge_attr, we, be.reshape(1, H), wcat)


def _node_pre_body(x_ref, win_ref, bin_ref, wt_ref, bt_ref,
                   h_ref, td_ref, ts_ref):
    h = _leaky(jnp.dot(x_ref[...], win_ref[...],
                       preferred_element_type=jnp.float32) + bin_ref[...])
    h_ref[...] = h
    t = jnp.dot(h, wt_ref[...], preferred_element_type=jnp.float32) + bt_ref[...]
    w = _pack_full(t)
    td_ref[...] = w[:, :32]
    ts_ref[...] = w[:, 32:]


def _node_pre(x, win, bin_, wt, bt):
    BN = 2000
    grid = N // BN
    return pl.pallas_call(
        _node_pre_body,
        grid=(grid,),
        in_specs=[
            pl.BlockSpec((BN, D), lambda i: (i, 0)),
            pl.BlockSpec((D, H), lambda i: (0, 0)),
            pl.BlockSpec((1, H), lambda i: (0, 0)),
            pl.BlockSpec((H, 4 * H), lambda i: (0, 0)),
            pl.BlockSpec((1, 4 * H), lambda i: (0, 0)),
        ],
        out_specs=[
            pl.BlockSpec((BN, H), lambda i: (i, 0)),
            pl.BlockSpec((BN, H), lambda i: (i, 0)),
            pl.BlockSpec((BN, H), lambda i: (i, 0)),
        ],
        out_shape=[
            jax.ShapeDtypeStruct((N, H), jnp.float32),
            jax.ShapeDtypeStruct((N, H), jnp.uint32),
            jax.ShapeDtypeStruct((N, H), jnp.uint32),
        ],
    )(x, win, bin_.reshape(1, H), wt, bt.reshape(1, 4 * H))


def _node_mid_body(h_ref, agg_ref, wt_ref, bt_ref, h1_ref, td_ref, ts_ref):
    h1 = h_ref[...] + agg_ref[0] + agg_ref[1]
    h1_ref[...] = h1
    t = jnp.dot(h1, wt_ref[...], preferred_element_type=jnp.float32) + bt_ref[...]
    w = _pack_full(t)
    td_ref[...] = w[:, :32]
    ts_ref[...] = w[:, 32:]


def _node_mid(h, agg, wt, bt):
    BN = 2000
    grid = N // BN
    return pl.pallas_call(
        _node_mid_body,
        grid=(grid,),
        in_specs=[
            pl.BlockSpec((BN, H), lambda i: (i, 0)),
            pl.BlockSpec((NC, BN, H), lambda i: (0, i, 0)),
            pl.BlockSpec((H, 4 * H), lambda i: (0, 0)),
            pl.BlockSpec((1, 4 * H), lambda i: (0, 0)),
        ],
        out_specs=[
            pl.BlockSpec((BN, H), lambda i: (i, 0)),
            pl.BlockSpec((BN, H), lambda i: (i, 0)),
            pl.BlockSpec((BN, H), lambda i: (i, 0)),
        ],
        out_shape=[
            jax.ShapeDtypeStruct((N, H), jnp.float32),
            jax.ShapeDtypeStruct((N, H), jnp.uint32),
            jax.ShapeDtypeStruct((N, H), jnp.uint32),
        ],
    )(h, agg, wt, bt.reshape(1, 4 * H))


def _node_out_body(h_ref, agg_ref, wout_ref, bout_ref, out_ref):
    h2 = h_ref[...] + agg_ref[0] + agg_ref[1]
    out_ref[...] = jnp.dot(h2, wout_ref[...],
                           preferred_element_type=jnp.float32) + bout_ref[...]


def _node_out(h, agg, wout, bout):
    BN = 2000
    grid = N // BN
    return pl.pallas_call(
        _node_out_body,
        grid=(grid,),
        in_specs=[
            pl.BlockSpec((BN, H), lambda i: (i, 0)),
            pl.BlockSpec((NC, BN, H), lambda i: (0, i, 0)),
            pl.BlockSpec((H, O), lambda i: (0, 0)),
            pl.BlockSpec((1, O), lambda i: (0, 0)),
        ],
        out_specs=pl.BlockSpec((BN, O), lambda i: (i, 0)),
        out_shape=jax.ShapeDtypeStruct((N, O), jnp.float32),
    )(h, agg, wout, bout.reshape(1, O))


def _edge_pass_body(td_hbm, ts_hbm, ep_hbm, idx_hbm, out_hbm,
                    idx_v, gtd_v, gts_v, ge_v, m_v, z_v, agg_sh,
                    sem_td0, sem_td1, sem_ts0, sem_ts1, sem_e0, sem_e1,
                    sem_m0, sem_m1):
    cid = lax.axis_index("c")
    sid = lax.axis_index("s")
    wid = sid * NC + cid
    epq = wid // 8
    epr = (wid % 8) * CPW
    sems = ((sem_td0, sem_ts0, sem_e0), (sem_td1, sem_ts1, sem_e1))
    sems_m = (sem_m0, sem_m1)

    pltpu.sync_copy(idx_hbm.at[0, pl.ds(wid * CPW, CPW)], idx_v.at[0])
    pltpu.sync_copy(idx_hbm.at[1, pl.ds(wid * CPW, CPW)], idx_v.at[1])

    zero16 = jnp.zeros((16,), jnp.float32)

    def zero_body(i, _):
        z_v[i, pl.ds(0, 16)] = zero16
        z_v[i, pl.ds(16, 16)] = zero16
        return 0

    lax.fori_loop(0, CR, zero_body, 0, unroll=4)
    pltpu.sync_copy(z_v, agg_sh.at[pl.ds(sid * CR, CR)])

    @pl.when(sid == 0)
    def _():
        pltpu.sync_copy(z_v.at[pl.ds(0, CR_TAIL)],
                        agg_sh.at[pl.ds(NS * CR, CR_TAIL)])

    plsc.subcore_barrier()

    def start(c, slot):
        std, sts, se = sems[slot]
        pltpu.async_copy(td_hbm.at[idx_v.at[1, c]], gtd_v.at[slot], std)
        pltpu.async_copy(ts_hbm.at[idx_v.at[0, c]], gts_v.at[slot], sts)
        pltpu.async_copy(ep_hbm.at[pl.ds((epr + c) * CHUNK, CHUNK),
                                   pl.ds(32 * epq, 32)], ge_v.at[slot], se)

    def wait(slot):
        std, sts, se = sems[slot]
        pltpu.make_async_copy(td_hbm.at[idx_v.at[1, 0]], gtd_v.at[slot],
                              std).wait()
        pltpu.make_async_copy(ts_hbm.at[idx_v.at[0, 0]], gts_v.at[slot],
                              sts).wait()
        pltpu.make_async_copy(ep_hbm.at[pl.ds(0, CHUNK), pl.ds(0, 32)],
                              ge_v.at[slot], se).wait()

    def wait_scatter(slot):
        pltpu.make_async_copy(m_v.at[slot], agg_sh.at[idx_v.at[1, 0]],
                              sems_m[slot]).wait()

    def compute_scatter(c, slot):
        td, ts, e = gtd_v.at[slot], gts_v.at[slot], ge_v.at[slot]
        m = m_v.at[slot]
        unp = functools.partial(plsc.unpack, format=plsc.PackFormat.INTERLEAVED)

        def unp16(ref, i, off):
            w = ref[i, pl.ds(off, 16)]
            return unp(plsc.bitcast(w, jnp.bfloat16))

        @pl.when(c >= 2)
        def _():
            wait_scatter(slot)

        @plsc.parallel_loop(0, CHUNK, unroll=4)
        def _(i):
            fd0, fd1 = unp16(td, i, 0)
            sd0, sd1 = unp16(td, i, 16)
            fs0, fs1 = unp16(ts, i, 0)
            ss0, ss1 = unp16(ts, i, 16)
            fe0, fe1 = unp16(e, i, 0)
            se0, se1 = unp16(e, i, 16)
            m[i, pl.ds(0, 16)] = _gate(fd0 + fs0 + fe0, sd0 + ss0 + se0)
            m[i, pl.ds(16, 16)] = _gate(fd1 + fs1 + fe1, sd1 + ss1 + se1)

        pltpu.async_copy(m, agg_sh.at[idx_v.at[1, c]], sems_m[slot], add=True)

    start(0, 0)

    def pipe_body(jj, _):
        c0 = 2 * jj
        c1 = c0 + 1
        start(c1, 1)
        wait(0)
        compute_scatter(c0, 0)

        @pl.when(c1 + 1 < CPW)
        def _():
            start(c1 + 1, 0)

        wait(1)
        compute_scatter(c1, 1)
        return 0

    lax.fori_loop(0, CPW // 2, pipe_body, 0)
    wait_scatter(0)
    wait_scatter(1)
    plsc.subcore_barrier()

    pltpu.sync_copy(agg_sh.at[pl.ds(sid * CR, CR)], z_v)
    pltpu.sync_copy(z_v, out_hbm.at[cid, pl.ds(sid * CR, CR)])

    @pl.when(sid == 0)
    def _():
        pltpu.sync_copy(agg_sh.at[pl.ds(NS * CR, OUT_TAIL)],
                        z_v.at[pl.ds(0, OUT_TAIL)])
        pltpu.sync_copy(z_v.at[pl.ds(0, OUT_TAIL)],
                        out_hbm.at[cid, pl.ds(NS * CR, OUT_TAIL)])


def _edge_pass(td_pad, ts_pad, ep, idx):
    mesh = plsc.VectorSubcoreMesh(core_axis_name="c", subcore_axis_name="s",
                                  num_cores=NC, num_subcores=NS)
    f = functools.partial(
        pl.kernel,
        out_type=jax.ShapeDtypeStruct((NC, N, H), jnp.float32),
        mesh=mesh,
        scratch_types=[
            pltpu.VMEM((2, CPW, CHUNK), jnp.int32),
            pltpu.VMEM((2, CHUNK, H), jnp.uint32),
            pltpu.VMEM((2, CHUNK, H), jnp.uint32),
            pltpu.VMEM((2, CHUNK, H), jnp.uint32),
            pltpu.VMEM((2, CHUNK, H), jnp.float32),
            pltpu.VMEM((CR, H), jnp.float32),
            pltpu.VMEM_SHARED((N_AGG, H), jnp.float32),
            pltpu.SemaphoreType.DMA,
            pltpu.SemaphoreType.DMA,
            pltpu.SemaphoreType.DMA,
            pltpu.SemaphoreType.DMA,
            pltpu.SemaphoreType.DMA,
            pltpu.SemaphoreType.DMA,
            pltpu.SemaphoreType.DMA,
            pltpu.SemaphoreType.DMA,
        ],
        compiler_params=pltpu.CompilerParams(use_tc_tiling_on_sc=False,
                                             needs_layout_passes=False),
    )(_edge_pass_body)
    return f(td_pad, ts_pad, ep, idx)


def _fold_bn(w, b, g, be, m, v, eps=1e-5):
    scale = g / jnp.sqrt(v + eps)
    return w * scale[None, :], (b - m) * scale + be


def kernel(x, edge_index, edge_attr, W_in, b_in, g_in, be_in, m_in, v_in,
           W_e, b_e, g_e, be_e, m_e, v_e, Wf1, bf1, Ws1, bs1, Wf2, bf2,
           Ws2, bs2, W_out, b_out):
    win, bin_ = _fold_bn(W_in, b_in, g_in, be_in, m_in, v_in)
    we, be = _fold_bn(W_e, b_e, g_e, be_e, m_e, v_e)

    def _lohi(*blocks):
        return jnp.concatenate([b[:, :16] for b in blocks]
                               + [b[:, 16:] for b in blocks], axis=1)

    def _lohi_b(*vecs):
        return jnp.concatenate([v[:16] for v in vecs]
                               + [v[16:] for v in vecs])

    z32 = jnp.zeros((H,), jnp.float32)
    wt1 = _lohi(Wf1[0:H], Ws1[0:H], Wf1[H:2 * H], Ws1[H:2 * H])
    bt1 = _lohi_b(bf1, bs1, z32, z32)
    wt2 = _lohi(Wf2[0:H], Ws2[0:H], Wf2[H:2 * H], Ws2[H:2 * H])
    bt2 = _lohi_b(bf2, bs2, z32, z32)
    wcat = _lohi(Wf1[2 * H:], Ws1[2 * H:], Wf2[2 * H:], Ws2[2 * H:])

    src = edge_index[0]
    dst = edge_index[1]
    pad = E_PAD - E
    src_p = jnp.concatenate([src, jnp.zeros((pad,), jnp.int32)])
    dst_p = jnp.concatenate([dst, jnp.full((pad,), N, jnp.int32)])
    idx = jnp.concatenate([src_p, dst_p]).reshape(2, NCHUNKS, CHUNK)

    ep1, ep2 = _edge_pre(edge_attr, we, be, wcat)

    h0, td1, ts1 = _node_pre(x, win, bin_, wt1, bt1)

    pad_t = ((0, N_TAB - N), (0, 0))
    agg1 = _edge_pass(jnp.pad(td1, pad_t), jnp.pad(ts1, pad_t), ep1, idx)

    h1, td2, ts2 = _node_mid(h0, agg1, wt2, bt2)

    agg2 = _edge_pass(jnp.pad(td2, pad_t), jnp.pad(ts2, pad_t), ep2, idx)

    return _node_out(h1, agg2, W_out, b_out)
